# baseline (speedup 1.0000x reference)
import functools

import jax
import jax.numpy as jnp
from jax import lax
from jax.experimental import pallas as pl
from jax.experimental.pallas import tpu as pltpu
from jax.experimental.pallas import tpu_sc as plsc

N = 10000
E = 320000
D = 128
H = 5
C = 4
L = 16
NW = 32
NP = 10240
EPW = E // NW
NPW = NP // NW

_SC_PARAMS = pltpu.CompilerParams(
    needs_layout_passes=False, use_tc_tiling_on_sc=False
)


@functools.cache
def _mesh():
    return plsc.VectorSubcoreMesh(
        core_axis_name="c", subcore_axis_name="s", num_cores=2, num_subcores=16
    )


def _wid():
    return lax.axis_index("s") * 2 + lax.axis_index("c")


def _deg_body(dst_hbm, out_hbm, dst_v, acc_v, sem):
    wid = _wid()
    cp = pltpu.async_copy(dst_hbm.at[pl.ds(wid * EPW, EPW)], dst_v, sem)
    zeros = jnp.zeros((L,), jnp.float32)

    @plsc.parallel_loop(0, NP, step=L, unroll=4)
    def zbody(i):
        acc_v[pl.ds(i, L)] = zeros

    cp.wait()
    ones = jnp.ones((L,), jnp.float32)

    @plsc.parallel_loop(0, EPW, step=L, unroll=8)
    def ebody(i):
        didx = dst_v[pl.ds(i, L)]
        plsc.addupdate_scatter(acc_v, [didx], ones)

    pltpu.sync_copy(acc_v, out_hbm.at[wid])


@functools.cache
def _deg_call():
    return pl.kernel(
        _deg_body,
        out_type=jax.ShapeDtypeStruct((NW, NP), jnp.float32),
        mesh=_mesh(),
        compiler_params=_SC_PARAMS,
        scratch_types=[
            pltpu.VMEM((EPW,), jnp.int32),
            pltpu.VMEM((NP,), jnp.float32),
            pltpu.SemaphoreType.DMA,
        ],
    )


def _dense1_body(parts_ref, xt_ref, w_ref, g_ref):
    deg = jnp.sum(parts_ref[...], axis=0, keepdims=True) + 1.0
    dinv = lax.rsqrt(deg)
    g = jnp.dot(w_ref[...], xt_ref[...], preferred_element_type=jnp.float32,
                precision=jax.lax.Precision.HIGHEST)
    g_ref[...] = jnp.concatenate([g[:H] * dinv, dinv], axis=0)


def _dense1_call(parts, xt, w1p):
    return pl.pallas_call(
        _dense1_body,
        out_shape=jax.ShapeDtypeStruct((H + 1, NP), jnp.float32),
    )(parts, xt, w1p)


def _scatter_body(g_hbm, src_hbm, dst_hbm, out_hbm, *refs):
    g_vs = refs[:H]
    acc_vs = refs[H:2 * H]
    src_v, dst_v, sem1, sem2 = refs[2 * H:]
    wid = _wid()
    base = wid * EPW
    cp1 = pltpu.async_copy(src_hbm.at[pl.ds(base, EPW)], src_v, sem1)
    cp2 = pltpu.async_copy(dst_hbm.at[pl.ds(base, EPW)], dst_v, sem2)
    for p in range(H):
        pltpu.sync_copy(g_hbm.at[p], g_vs[p])
    zeros = jnp.zeros((L,), jnp.float32)

    @plsc.parallel_loop(0, NP, step=L, unroll=4)
    def zbody(i):
        for p in range(H):
            acc_vs[p][pl.ds(i, L)] = zeros

    cp1.wait()
    cp2.wait()

    @plsc.parallel_loop(0, EPW, step=L, unroll=8)
    def ebody(i):
        sidx = src_v[pl.ds(i, L)]
        didx = dst_v[pl.ds(i, L)]
        for p in range(H):
            v = plsc.load_gather(g_vs[p], [sidx])
            plsc.addupdate_scatter(acc_vs[p], [didx], v)

    for p in range(H):
        pltpu.sync_copy(acc_vs[p], out_hbm.at[wid, p])


@functools.cache
def _make_scatter():
    return pl.kernel(
        _scatter_body,
        out_type=jax.ShapeDtypeStruct((NW, H, NP), jnp.float32),
        mesh=_mesh(),
        compiler_params=_SC_PARAMS,
        scratch_types=(
            [pltpu.VMEM((NP,), jnp.float32) for _ in range(2 * H)]
            + [
                pltpu.VMEM((EPW,), jnp.int32),
                pltpu.VMEM((EPW,), jnp.int32),
                pltpu.SemaphoreType.DMA,
                pltpu.SemaphoreType.DMA,
            ]
        ),
    )


def _make_reduce(final):
    n_out = C if final else H

    def body(parts_hbm, g_hbm, cb_hbm, out_hbm, acc_v, pbuf_v, cb_v, outb_v):
        wid = _wid()
        nb = wid * NPW
        pltpu.sync_copy(g_hbm.at[:, pl.ds(nb, NPW)], acc_v)
        pltpu.sync_copy(cb_hbm, cb_v)
        pltpu.sync_copy(parts_hbm.at[:, :, pl.ds(nb, NPW)], pbuf_v)

        iota = lax.iota(jnp.int32, L)

        def gbody(i, _):
            dv = acc_v[H, pl.ds(i * L, L)]

            def tsum(t, carry):
                return tuple(
                    carry[p] + pbuf_v[t, p, pl.ds(i * L, L)] for p in range(H)
                )

            s = lax.fori_loop(
                0, NW, tsum,
                tuple(acc_v[p, pl.ds(i * L, L)] for p in range(H)),
                unroll=4,
            )
            r = []
            for j in range(H):
                o = s[j] * dv + cb_v[H * n_out + j]
                r.append(jnp.maximum(o, jnp.zeros((L,), jnp.float32)))
            for k in range(n_out):
                h = r[0] * cb_v[k]
                for j in range(1, H):
                    h = h + r[j] * cb_v[j * n_out + k]
                if final:
                    h = h + cb_v[H * n_out + H + k]
                    idx = (iota + i * L) * C + k
                    plsc.store_scatter(outb_v, [idx], h)
                else:
                    outb_v[k, pl.ds(i * L, L)] = h * dv
            if not final:
                outb_v[H, pl.ds(i * L, L)] = dv
            return 0

        lax.fori_loop(0, NPW // L, gbody, 0)
        if final:
            pltpu.sync_copy(outb_v, out_hbm.at[pl.ds(nb * C, NPW * C)])
        else:
            pltpu.sync_copy(outb_v, out_hbm.at[:, pl.ds(nb, NPW)])

    out_type = (jax.ShapeDtypeStruct((NP * C,), jnp.float32) if final
                else jax.ShapeDtypeStruct((H + 1, NP), jnp.float32))
    outb = (pltpu.VMEM((NPW * C,), jnp.float32) if final
            else pltpu.VMEM((H + 1, NPW), jnp.float32))
    ncb = H * n_out + H + (n_out if final else 0)
    return pl.kernel(
        body,
        out_type=out_type,
        mesh=_mesh(),
        compiler_params=_SC_PARAMS,
        scratch_types=[
            pltpu.VMEM((H + 1, NPW), jnp.float32),
            pltpu.VMEM((NW, H, NPW), jnp.float32),
            pltpu.VMEM((ncb, L), jnp.float32),
            outb,
        ],
    )


_make_reduce = functools.cache(_make_reduce)


def kernel(x, edge_index, W1, b1, W2, b2, Wl, bl):
    src = edge_index[0]
    dst = edge_index[1]
    xt = jnp.pad(x.T, ((0, 0), (0, NP - N)))
    w1p = jnp.zeros((8, D), jnp.float32).at[:H].set(W1.T)
    cb1 = jnp.broadcast_to(
        jnp.concatenate([W2.reshape(H * H), b1])[:, None], (H * H + H, L)
    )
    cb2 = jnp.broadcast_to(
        jnp.concatenate([Wl.reshape(H * C), b2, bl])[:, None],
        (H * C + H + C, L),
    )

    dparts = _deg_call()(dst)
    g1 = _dense1_call(dparts, xt, w1p)
    p1 = _make_scatter()(g1, src, dst)
    g2 = _make_reduce(False)(p1, g1, cb1)
    p2 = _make_scatter()(g2, src, dst)
    flat = _make_reduce(True)(p2, g2, cb2)
    return flat.reshape(NP, C)[:N]


# speedup vs baseline: 1.0166x; 1.0166x over previous
import functools

import jax
import jax.numpy as jnp
from jax import lax
from jax.experimental import pallas as pl
from jax.experimental.pallas import tpu as pltpu
from jax.experimental.pallas import tpu_sc as plsc

N = 10000
E = 320000
D = 128
H = 5
C = 4
L = 16
NW = 32
NP = 10240
EPW = E // NW
NPW = NP // NW

_SC_PARAMS = pltpu.CompilerParams(
    needs_layout_passes=False, use_tc_tiling_on_sc=False
)


@functools.cache
def _mesh():
    return plsc.VectorSubcoreMesh(
        core_axis_name="c", subcore_axis_name="s", num_cores=2, num_subcores=16
    )


def _wid():
    return lax.axis_index("s") * 2 + lax.axis_index("c")


def _deg_body(dst_hbm, out_hbm, dst_v, acc_v, sem):
    wid = _wid()
    cp = pltpu.async_copy(dst_hbm.at[pl.ds(wid * EPW, EPW)], dst_v, sem)
    zeros = jnp.zeros((L,), jnp.float32)

    @plsc.parallel_loop(0, NP, step=L, unroll=4)
    def zbody(i):
        acc_v[pl.ds(i, L)] = zeros

    cp.wait()
    ones = jnp.ones((L,), jnp.float32)

    @plsc.parallel_loop(0, EPW, step=L, unroll=8)
    def ebody(i):
        didx = dst_v[pl.ds(i, L)]
        plsc.addupdate_scatter(acc_v, [didx], ones)

    pltpu.sync_copy(acc_v, out_hbm.at[wid])


@functools.cache
def _deg_call():
    return pl.kernel(
        _deg_body,
        out_type=jax.ShapeDtypeStruct((NW, NP), jnp.float32),
        mesh=_mesh(),
        compiler_params=_SC_PARAMS,
        scratch_types=[
            pltpu.VMEM((EPW,), jnp.int32),
            pltpu.VMEM((NP,), jnp.float32),
            pltpu.SemaphoreType.DMA,
        ],
    )


def _dense1_body(parts_ref, x_ref, w_ref, g_ref):
    deg = jnp.sum(parts_ref[...], axis=0, keepdims=True) + 1.0
    dinv = lax.rsqrt(deg)
    g = jnp.dot(w_ref[...], x_ref[...].T, preferred_element_type=jnp.float32,
                precision=jax.lax.Precision.HIGHEST)
    gp = jnp.pad(g[:H], ((0, 0), (0, NP - N)))
    g_ref[...] = jnp.concatenate([gp * dinv, dinv], axis=0)


def _dense1_call(parts, xt, w1p):
    return pl.pallas_call(
        _dense1_body,
        out_shape=jax.ShapeDtypeStruct((H + 1, NP), jnp.float32),
    )(parts, xt, w1p)


def _scatter_body(g_hbm, src_hbm, dst_hbm, out_hbm, *refs):
    g_vs = refs[:H]
    acc_vs = refs[H:2 * H]
    src_v, dst_v, sem1, sem2 = refs[2 * H:]
    wid = _wid()
    base = wid * EPW
    cp1 = pltpu.async_copy(src_hbm.at[pl.ds(base, EPW)], src_v, sem1)
    cp2 = pltpu.async_copy(dst_hbm.at[pl.ds(base, EPW)], dst_v, sem2)
    for p in range(H):
        pltpu.sync_copy(g_hbm.at[p], g_vs[p])
    zeros = jnp.zeros((L,), jnp.float32)

    @plsc.parallel_loop(0, NP, step=L, unroll=4)
    def zbody(i):
        for p in range(H):
            acc_vs[p][pl.ds(i, L)] = zeros

    cp1.wait()
    cp2.wait()

    @plsc.parallel_loop(0, EPW, step=L, unroll=4)
    def ebody(i):
        sidx = src_v[pl.ds(i, L)]
        didx = dst_v[pl.ds(i, L)]
        for p in range(H):
            v = plsc.load_gather(g_vs[p], [sidx])
            plsc.addupdate_scatter(acc_vs[p], [didx], v)

    for p in range(H):
        pltpu.sync_copy(acc_vs[p], out_hbm.at[wid, p])


@functools.cache
def _make_scatter():
    return pl.kernel(
        _scatter_body,
        out_type=jax.ShapeDtypeStruct((NW, H, NP), jnp.float32),
        mesh=_mesh(),
        compiler_params=_SC_PARAMS,
        scratch_types=(
            [pltpu.VMEM((NP,), jnp.float32) for _ in range(2 * H)]
            + [
                pltpu.VMEM((EPW,), jnp.int32),
                pltpu.VMEM((EPW,), jnp.int32),
                pltpu.SemaphoreType.DMA,
                pltpu.SemaphoreType.DMA,
            ]
        ),
    )


def _make_reduce(final):
    n_out = C if final else H

    def body(parts_hbm, g_hbm, cb_hbm, out_hbm, acc_v, pbuf_v, cb_v, outb_v):
        wid = _wid()
        nb = wid * NPW
        pltpu.sync_copy(g_hbm.at[:, pl.ds(nb, NPW)], acc_v)
        pltpu.sync_copy(cb_hbm, cb_v)
        pltpu.sync_copy(parts_hbm.at[:, :, pl.ds(nb, NPW)], pbuf_v)

        iota = lax.iota(jnp.int32, L)

        def gbody(i, _):
            dv = acc_v[H, pl.ds(i * L, L)]

            def tsum(t, carry):
                return tuple(
                    carry[p] + pbuf_v[t, p, pl.ds(i * L, L)] for p in range(H)
                )

            s = lax.fori_loop(
                0, NW, tsum,
                tuple(acc_v[p, pl.ds(i * L, L)] for p in range(H)),
                unroll=4,
            )
            r = []
            for j in range(H):
                o = s[j] * dv + cb_v[H * n_out + j]
                r.append(jnp.maximum(o, jnp.zeros((L,), jnp.float32)))
            for k in range(n_out):
                h = r[0] * cb_v[k]
                for j in range(1, H):
                    h = h + r[j] * cb_v[j * n_out + k]
                if final:
                    h = h + cb_v[H * n_out + H + k]
                    idx = (iota + i * L) * C + k
                    plsc.store_scatter(outb_v, [idx], h)
                else:
                    outb_v[k, pl.ds(i * L, L)] = h * dv
            if not final:
                outb_v[H, pl.ds(i * L, L)] = dv
            return 0

        lax.fori_loop(0, NPW // L, gbody, 0)
        if final:
            pltpu.sync_copy(outb_v, out_hbm.at[pl.ds(nb * C, NPW * C)])
        else:
            pltpu.sync_copy(outb_v, out_hbm.at[:, pl.ds(nb, NPW)])

    out_type = (jax.ShapeDtypeStruct((NP * C,), jnp.float32) if final
                else jax.ShapeDtypeStruct((H + 1, NP), jnp.float32))
    outb = (pltpu.VMEM((NPW * C,), jnp.float32) if final
            else pltpu.VMEM((H + 1, NPW), jnp.float32))
    ncb = H * n_out + H + (n_out if final else 0)
    return pl.kernel(
        body,
        out_type=out_type,
        mesh=_mesh(),
        compiler_params=_SC_PARAMS,
        scratch_types=[
            pltpu.VMEM((H + 1, NPW), jnp.float32),
            pltpu.VMEM((NW, H, NPW), jnp.float32),
            pltpu.VMEM((ncb, L), jnp.float32),
            outb,
        ],
    )


_make_reduce = functools.cache(_make_reduce)


def kernel(x, edge_index, W1, b1, W2, b2, Wl, bl):
    src = edge_index[0]
    dst = edge_index[1]
    w1p = jnp.zeros((8, D), jnp.float32).at[:H].set(W1.T)
    cb1 = jnp.broadcast_to(
        jnp.concatenate([W2.reshape(H * H), b1])[:, None], (H * H + H, L)
    )
    cb2 = jnp.broadcast_to(
        jnp.concatenate([Wl.reshape(H * C), b2, bl])[:, None],
        (H * C + H + C, L),
    )

    dparts = _deg_call()(dst)
    g1 = _dense1_call(dparts, x, w1p)
    p1 = _make_scatter()(g1, src, dst)
    g2 = _make_reduce(False)(p1, g1, cb1)
    p2 = _make_scatter()(g2, src, dst)
    flat = _make_reduce(True)(p2, g2, cb2)
    return flat.reshape(NP, C)[:N]
